# Initial kernel scaffold; baseline (speedup 1.0000x reference)
import jax
import jax.numpy as jnp
from jax.experimental import pallas as pl


def kernel(vectors, span_starts, span_ends, Wm1, bm1, Wm2, bm2, Wp1, bp1, Wp2, bp2):
    raise NotImplementedError("write your pallas kernel here")


# speedup vs baseline: 9.5921x; 9.5921x over previous
import functools

import jax
import jax.numpy as jnp
from jax.experimental import pallas as pl
from jax.experimental.pallas import tpu as pltpu

_P_LAMBDA = 0.4
_K = 50


def _mention_body(v_ref, wm1_ref, bm1_ref, wm2_ref, bm2_ref, out_ref):
    h = jnp.maximum(
        jnp.dot(v_ref[...], wm1_ref[...], preferred_element_type=jnp.float32)
        + bm1_ref[...], 0.0)
    out_ref[...] = (
        jnp.dot(h, wm2_ref[...], preferred_element_type=jnp.float32)
        + bm2_ref[...])


def _pair_body(n, ni, vp_ref, sp_ref, wa_ref, wb_ref, wc_ref, bp1_ref,
               wp2_ref, bp2_ref, out_ref, sc_ref):
    vp = vp_ref[...]
    sp = sp_ref[...]
    a = jnp.dot(vp, wa_ref[...], preferred_element_type=jnp.float32)
    b = jnp.dot(vp, wb_ref[...], preferred_element_type=jnp.float32)
    ai = a[:ni] + bp1_ref[...]
    vi = vp[:ni]
    base = sp[:ni] + bp2_ref[0, 0]
    wc = wc_ref[...]
    wp2 = wp2_ref[...]
    row = jax.lax.broadcasted_iota(jnp.int32, (ni, 1), 0)
    for k in range(1, _K + 1):
        vj = vp[k:k + ni]
        bj = b[k:k + ni]
        sj = sp[k:k + ni]
        h = jnp.maximum(
            ai + bj + jnp.dot(vi * vj, wc, preferred_element_type=jnp.float32),
            0.0)
        ps = jnp.dot(h, wp2, preferred_element_type=jnp.float32)
        col = ps + sj + base
        col = jnp.where(row < n - k, col, -1e9)
        sc_ref[:, k - 1:k] = col
    sc_ref[:, _K:_K + 1] = jnp.zeros((ni, 1), jnp.float32)
    sc = sc_ref[...]
    mx = jnp.max(sc, axis=1, keepdims=True)
    e = jnp.exp(sc - mx)
    out_ref[...] = (e / jnp.sum(e, axis=1, keepdims=True))[:n - 1]


def kernel(vectors, span_starts, span_ends, Wm1, bm1, Wm2, bm2, Wp1, bp1,
           Wp2, bp2):
    t, d = vectors.shape
    n = int(_P_LAMBDA * t)
    hidden = Wm1.shape[1]

    mscores = pl.pallas_call(
        _mention_body,
        out_shape=jax.ShapeDtypeStruct((t, 1), jnp.float32),
    )(vectors, Wm1, bm1.reshape(1, hidden), Wm2, bm2.reshape(1, 1))[:, 0]

    order = jnp.argsort(-mscores)
    top = order[:n]
    skey = span_starts[top] * 100000 + span_ends[top]
    perm = jnp.argsort(-skey)
    idx = top[perm]
    v = vectors[idx]
    s = mscores[idx]

    ni = ((n - 1) + 7) // 8 * 8
    npad = (ni + _K + 7) // 8 * 8
    vp = jnp.zeros((npad, d), jnp.float32).at[:n].set(v)
    sp = jnp.zeros((npad, 1), jnp.float32).at[:n, 0].set(s)
    wa, wb, wc = Wp1[:d], Wp1[d:2 * d], Wp1[2 * d:]

    probs = pl.pallas_call(
        functools.partial(_pair_body, n, ni),
        out_shape=jax.ShapeDtypeStruct((n - 1, _K + 1), jnp.float32),
        scratch_shapes=[pltpu.VMEM((ni, _K + 1), jnp.float32)],
    )(vp, sp, wa, wb, wc, bp1.reshape(1, hidden), Wp2, bp2.reshape(1, 1))
    return probs
